# baseline (speedup 1.0000x reference)
import functools

import jax
import jax.numpy as jnp
from jax import lax
from jax.experimental import pallas as pl
from jax.experimental.pallas import tpu as pltpu
from jax.experimental.pallas import tpu_sc as plsc

_N = 10000
_E = 320000
_D = 128
_ED = 16

_NC = 2
_NS = 16
_NW = _NC * _NS
_CH = 128
_CHUNKS = 80
_EPW = _CH * _CHUNKS
_EPAD = _NW * _EPW
_NROWS = _EPAD // _CH
_NPAD = 10240
_RPT = _NPAD // _NS

_mesh = plsc.VectorSubcoreMesh(core_axis_name="c", subcore_axis_name="s")


def _seg_sum_h_body(feat_hbm, src_hbm, dst_hbm, zeros_hbm, out_hbm,
                    sidx0, sidx1, didx, rows0, rows1, sem_i, sem_g, sem_s,
                    acc_sh):
    c = lax.axis_index("c")
    s = lax.axis_index("s")
    wid = c * _NS + s
    r0 = wid * _CHUNKS

    pltpu.sync_copy(dst_hbm.at[pl.ds(r0, _CHUNKS)], didx)

    pltpu.sync_copy(zeros_hbm, rows0)
    for k in range(_RPT // _CH):
        pltpu.sync_copy(rows0, acc_sh.at[pl.ds(s * _RPT + k * _CH, _CH)])
    plsc.subcore_barrier()

    rows = (rows0, rows1)
    sidx = (sidx0, sidx1)

    def wait_g():
        pltpu.make_async_copy(zeros_hbm, rows0, sem_g).wait()

    def wait_s():
        pltpu.make_async_copy(zeros_hbm, rows0, sem_s).wait()

    def wait_i():
        pltpu.make_async_copy(src_hbm.at[pl.ds(0, 1)], sidx0, sem_i).wait()

    def load_idx(k, b):
        pltpu.async_copy(src_hbm.at[pl.ds(r0 + k, 1)], sidx[b], sem_i)

    def gather(k, b):
        pltpu.async_copy(feat_hbm.at[sidx[b].at[0]], rows[b], sem_g)

    def scat(k, b):
        pltpu.async_copy(rows[b], acc_sh.at[didx.at[k]], sem_s, add=True)

    load_idx(0, 0)
    wait_i()
    gather(0, 0)
    load_idx(1, 1)
    wait_g()
    scat(0, 0)
    wait_i()
    gather(1, 1)
    load_idx(2, 0)

    def body(i, carry):
        for b in (0, 1):
            k = 1 + 2 * i + b
            wait_g()
            scat(k, 1 - b)
            wait_s()
            wait_i()
            gather(k + 1, b)
            load_idx(k + 2, 1 - b)
        return carry

    lax.fori_loop(0, (_CHUNKS - 4) // 2, body, 0)

    k = _CHUNKS - 3
    wait_g()
    scat(k, 1)
    wait_s()
    wait_i()
    gather(k + 1, 0)
    load_idx(k + 2, 1)
    wait_g()
    scat(k + 1, 0)
    wait_s()
    wait_i()
    gather(k + 2, 1)
    wait_g()
    scat(k + 2, 1)
    wait_s()
    wait_s()
    plsc.subcore_barrier()

    for k in range(_RPT // _CH):
        rr = s * _RPT + k * _CH
        pltpu.sync_copy(acc_sh.at[pl.ds(rr, _CH)], rows0)
        pltpu.sync_copy(rows0, out_hbm.at[c, pl.ds(rr, _CH)])


_seg_sum_h = pl.kernel(
    _seg_sum_h_body,
    out_type=jax.ShapeDtypeStruct((_NC, _NPAD, _D), jnp.float32),
    mesh=_mesh,
    compiler_params=pltpu.CompilerParams(use_tc_tiling_on_sc=True),
    scratch_types=[
        pltpu.VMEM((1, _CH), jnp.int32),
        pltpu.VMEM((1, _CH), jnp.int32),
        pltpu.VMEM((_CHUNKS, _CH), jnp.int32),
        pltpu.VMEM((_CH, _D), jnp.float32),
        pltpu.VMEM((_CH, _D), jnp.float32),
        pltpu.SemaphoreType.DMA,
        pltpu.SemaphoreType.DMA,
        pltpu.SemaphoreType.DMA,
        pltpu.VMEM_SHARED((_NPAD, _D), jnp.float32),
    ],
)


def _seg_sum_ef_body(feat_hbm, src_hbm, dst_hbm, zeros_hbm, out_hbm,
                     didx, rows, sem, acc_sh):
    c = lax.axis_index("c")
    s = lax.axis_index("s")
    wid = c * _NS + s
    base = wid * _EPW
    r0 = wid * _CHUNKS

    pltpu.sync_copy(dst_hbm.at[pl.ds(r0, _CHUNKS)], didx)
    pltpu.sync_copy(zeros_hbm, rows)
    for k in range(_RPT // _CH):
        pltpu.sync_copy(rows, acc_sh.at[pl.ds(s * _RPT + k * _CH, _CH)])
    plsc.subcore_barrier()

    def body(i, carry):
        off = base + i * _CH
        pltpu.sync_copy(feat_hbm.at[pl.ds(off, _CH)], rows)
        pltpu.sync_copy(rows, acc_sh.at[didx.at[i]], add=True)
        return carry

    lax.fori_loop(0, _CHUNKS, body, 0)
    plsc.subcore_barrier()

    for k in range(_RPT // _CH):
        rr = s * _RPT + k * _CH
        pltpu.sync_copy(acc_sh.at[pl.ds(rr, _CH)], rows)
        pltpu.sync_copy(rows, out_hbm.at[c, pl.ds(rr, _CH)])


_seg_sum_ef = pl.kernel(
    _seg_sum_ef_body,
    out_type=jax.ShapeDtypeStruct((_NC, _NPAD, 32), jnp.float32),
    mesh=_mesh,
    compiler_params=pltpu.CompilerParams(use_tc_tiling_on_sc=False),
    scratch_types=[
        pltpu.VMEM((_CHUNKS, _CH), jnp.int32),
        pltpu.VMEM((_CH, 32), jnp.float32),
        pltpu.SemaphoreType.DMA,
        pltpu.VMEM_SHARED((_NPAD, 32), jnp.float32),
    ],
)


def _heads_body(psrc_hbm, pdst_hbm, src_hbm, dst_hbm, out_hbm,
                sidx, didx, acc, sem):
    c = lax.axis_index("c")
    s = lax.axis_index("s")
    wid = c * _NS + s
    base = wid * _EPW
    r0 = wid * _CHUNKS

    pltpu.sync_copy(src_hbm.at[pl.ds(r0, _CHUNKS)], sidx)
    pltpu.sync_copy(dst_hbm.at[pl.ds(r0, _CHUNKS)], didx)

    def body(i, carry):
        pltpu.async_copy(psrc_hbm.at[sidx.at[i]], acc, sem).wait()
        pltpu.async_copy(pdst_hbm.at[didx.at[i]], acc, sem, add=True).wait()
        pltpu.sync_copy(acc, out_hbm.at[pl.ds(base + i * _CH, _CH)])
        return carry

    lax.fori_loop(0, _CHUNKS, body, 0)


_heads = pl.kernel(
    _heads_body,
    out_type=jax.ShapeDtypeStruct((_EPAD, 16), jnp.float32),
    mesh=_mesh,
    compiler_params=pltpu.CompilerParams(use_tc_tiling_on_sc=False),
    scratch_types=[
        pltpu.VMEM((_CHUNKS, _CH), jnp.int32),
        pltpu.VMEM((_CHUNKS, _CH), jnp.int32),
        pltpu.VMEM((_CH, 16), jnp.float32),
        pltpu.SemaphoreType.DMA,
    ],
)


def _dense_body(h_ref, s0_ref, s1_ref, t0_ref, t1_ref,
                wm1_ref, wmaug_ref, wa1_ref, wa2_ref, ba_ref, out_ref):
    f32 = jnp.float32
    S = s0_ref[...] + s1_ref[...]
    T = t0_ref[...] + t1_ref[...]
    hn = jnp.dot(S, wm1_ref[...], preferred_element_type=f32)
    hn = hn + jnp.dot(T, wmaug_ref[...], preferred_element_type=f32)
    acc = jnp.dot(h_ref[...], wa1_ref[...], preferred_element_type=f32)
    acc = acc + jnp.dot(hn, wa2_ref[...], preferred_element_type=f32)
    acc = acc + ba_ref[...]
    out_ref[...] = jnp.where(acc >= 0, acc, 0.01 * acc)


_BR = 512


def _tc_dense(h, S0, S1, T0, T1, Wm1, WmAug, Wa1, Wa2, ba_row):
    full = lambda shape: pl.BlockSpec(shape, lambda i: (0, 0))
    row = lambda w: pl.BlockSpec((_BR, w), lambda i: (i, 0))
    return pl.pallas_call(
        _dense_body,
        grid=(_NPAD // _BR,),
        in_specs=[row(_D), row(_D), row(_D), row(32), row(32),
                  full((_D, _D)), full((32, _D)), full((_D, _D)),
                  full((_D, _D)), full((1, _D))],
        out_specs=row(_D),
        out_shape=jax.ShapeDtypeStruct((_NPAD, _D), jnp.float32),
    )(h, S0, S1, T0, T1, Wm1, WmAug, Wa1, Wa2, ba_row)


def _proj_body(h_ref, wsrc_ref, wdst_ref, bias_ref, psrc_ref, pdst_ref):
    f32 = jnp.float32
    h = h_ref[...]
    psrc_ref[...] = jnp.dot(h, wsrc_ref[...], preferred_element_type=f32)
    pdst_ref[...] = (jnp.dot(h, wdst_ref[...], preferred_element_type=f32)
                     + bias_ref[...])


def _tc_proj(h, Wsrc, Wdst, bias_row):
    full = lambda shape: pl.BlockSpec(shape, lambda i: (0, 0))
    row = lambda w: pl.BlockSpec((_BR, w), lambda i: (i, 0))
    return pl.pallas_call(
        _proj_body,
        grid=(_NPAD // _BR,),
        in_specs=[row(_D), full((_D, 16)), full((_D, 16)), full((1, 16))],
        out_specs=[row(16), row(16)],
        out_shape=[jax.ShapeDtypeStruct((_NPAD, 16), jnp.float32),
                   jax.ShapeDtypeStruct((_NPAD, 16), jnp.float32)],
    )(h, Wsrc, Wdst, bias_row)


def kernel(nfeats, efeats, edge_index, W1m, b1m, A1, W1a, b1a,
           W2m, b2m, A2, W2a, b2a, Wc, bc, Wf, bf):
    f32 = jnp.float32
    h0 = nfeats[:, 0, :]
    ef = efeats[:, 0, :]
    src = edge_index[0]
    dst = edge_index[1]

    pad_e = _EPAD - _E
    srcp = jnp.concatenate([src, jnp.zeros((pad_e,), jnp.int32)])
    srcp = srcp.reshape(_NROWS, _CH)
    dstp = jnp.concatenate([dst, jnp.full((pad_e,), _N, jnp.int32)])
    dstp = dstp.reshape(_NROWS, _CH)
    ef32 = jnp.concatenate(
        [ef, jnp.ones((_E, 1), f32), jnp.zeros((_E, 15), f32)], axis=1)
    ef32 = jnp.concatenate([ef32, jnp.zeros((pad_e, 32), f32)], axis=0)
    h0p = jnp.concatenate([h0, jnp.zeros((_NPAD - _N, _D), f32)], axis=0)
    z128 = jnp.zeros((_CH, _D), f32)
    z32 = jnp.zeros((_CH, 32), f32)

    WmAug1 = jnp.concatenate([W1m[_D:], b1m[None], jnp.zeros((15, _D), f32)])
    WmAug2 = jnp.concatenate([W2m[_D:], b2m[None], jnp.zeros((15, _D), f32)])

    Taug = _seg_sum_ef(ef32, srcp, dstp, z32)
    S1 = _seg_sum_h(h0p, srcp, dstp, z128)
    h1 = _tc_dense(h0p, S1[0], S1[1], Taug[0], Taug[1],
                   W1m[:_D], WmAug1, W1a[:_D], W1a[_D:], b1a[None])
    S2 = _seg_sum_h(h1, srcp, dstp, z128)
    h2 = _tc_dense(h1, S2[0], S2[1], Taug[0], Taug[1],
                   W2m[:_D], WmAug2, W2a[:_D], W2a[_D:], b2a[None])

    Wsrc = jnp.concatenate([Wc[:_D], Wf[:_D], jnp.zeros((_D, 4), f32)], axis=1)
    Wdst = jnp.concatenate([Wc[_D:], Wf[_D:], jnp.zeros((_D, 4), f32)], axis=1)
    bias16 = jnp.concatenate([bc, bf, jnp.zeros((4,), f32)])[None]
    Psrc, Pdst = _tc_proj(h2, Wsrc, Wdst, bias16)

    out16 = _heads(Psrc, Pdst, srcp, dstp)
    coarse = out16[:_E, 0:2]
    fine = out16[:_E, 2:12]
    return coarse, fine


# speedup vs baseline: 1.9051x; 1.9051x over previous
import functools

import jax
import jax.numpy as jnp
from jax import lax
from jax.experimental import pallas as pl
from jax.experimental.pallas import tpu as pltpu
from jax.experimental.pallas import tpu_sc as plsc

_N = 10000
_E = 320000
_D = 128
_ED = 16

_NC = 2
_NS = 16
_NW = _NC * _NS
_CH = 80
_CHUNKS = 125
_EPW = _CH * _CHUNKS
_NROWS = _E // _CH
_NPAD = 10112
_RPT = _NPAD // _NS

_mesh = plsc.VectorSubcoreMesh(core_axis_name="c", subcore_axis_name="s")


def _acc_slices():
    full, tail = divmod(_RPT, _CH)
    out = [(k * _CH, _CH) for k in range(full)]
    if tail:
        out.append((full * _CH, tail))
    return out


def _seg_sum_h_body(feat_hbm, src_hbm, dst_hbm, zeros_hbm, out_hbm,
                    sa0, sa1, sa2, sa3, da0, da1, da2, da3,
                    rows0, rows1, rows2, rows3,
                    sem_i, sem_g, sem_s, acc_sh):
    c = lax.axis_index("c")
    s = lax.axis_index("s")
    wid = c * _NS + s
    r0 = wid * _CHUNKS

    rows = (rows0, rows1, rows2, rows3)
    sa = (sa0, sa1, sa2, sa3)
    da = (da0, da1, da2, da3)

    pltpu.sync_copy(zeros_hbm, rows0)
    for off, n in _acc_slices():
        pltpu.sync_copy(rows0.at[pl.ds(0, n)],
                        acc_sh.at[pl.ds(s * _RPT + off, n)])
    plsc.subcore_barrier()

    def wait_g():
        pltpu.make_async_copy(zeros_hbm, rows0, sem_g).wait()

    def wait_s():
        pltpu.make_async_copy(zeros_hbm, rows0, sem_s).wait()

    def wait_i():
        pltpu.make_async_copy(src_hbm.at[pl.ds(0, 1)], sa0, sem_i).wait()
        pltpu.make_async_copy(src_hbm.at[pl.ds(0, 1)], da0, sem_i).wait()

    def load_idx(k, j):
        pltpu.async_copy(src_hbm.at[pl.ds(r0 + k, 1)], sa[j], sem_i)
        pltpu.async_copy(dst_hbm.at[pl.ds(r0 + k, 1)], da[j], sem_i)

    def gather(k, j):
        pltpu.async_copy(feat_hbm.at[sa[j].at[0]], rows[j], sem_g)

    def scat(k, j):
        pltpu.async_copy(rows[j], acc_sh.at[da[j].at[0]], sem_s, add=True)

    def step(k, j, n_ahead, n_idx, first=False):
        wait_g()
        scat(k, j)
        if not first:
            wait_s()
        if n_ahead:
            wait_i()
            gather(k + 2, (j + 2) % 4)
        if n_idx:
            load_idx(k + 3, (j + 3) % 4)

    load_idx(0, 0)
    load_idx(1, 1)
    load_idx(2, 2)
    wait_i()
    gather(0, 0)
    wait_i()
    gather(1, 1)
    step(0, 0, True, True, first=True)
    step(1, 1, True, True)
    step(2, 2, True, True)

    def body(i, carry):
        for b4 in range(4):
            k = 3 + 4 * i + b4
            step(k, (3 + b4) % 4, True, True)
        return carry

    lax.fori_loop(0, (_CHUNKS - 9) // 4, body, 0)

    step(_CHUNKS - 6, (_CHUNKS - 6) % 4, True, True)
    step(_CHUNKS - 5, (_CHUNKS - 5) % 4, True, True)
    step(_CHUNKS - 4, (_CHUNKS - 4) % 4, True, True)
    step(_CHUNKS - 3, (_CHUNKS - 3) % 4, True, False)
    step(_CHUNKS - 2, (_CHUNKS - 2) % 4, False, False)
    step(_CHUNKS - 1, (_CHUNKS - 1) % 4, False, False)
    wait_s()
    plsc.subcore_barrier()

    for off, n in _acc_slices():
        rr = s * _RPT + off
        pltpu.sync_copy(acc_sh.at[pl.ds(rr, n)], rows0.at[pl.ds(0, n)])
        pltpu.sync_copy(rows0.at[pl.ds(0, n)], out_hbm.at[c, pl.ds(rr, n)])


_seg_sum_h = pl.kernel(
    _seg_sum_h_body,
    out_type=jax.ShapeDtypeStruct((_NC, _NPAD, _D), jnp.float32),
    mesh=_mesh,
    compiler_params=pltpu.CompilerParams(use_tc_tiling_on_sc=True),
    scratch_types=(
        [pltpu.VMEM((1, _CH), jnp.int32) for _ in range(8)]
        + [pltpu.VMEM((_CH, _D), jnp.float32) for _ in range(4)]
        + [pltpu.SemaphoreType.DMA] * 3
        + [pltpu.VMEM_SHARED((_NPAD, _D), jnp.float32)]
    ),
)


def _seg_sum_ef_body(feat_hbm, src_hbm, dst_hbm, zeros_hbm, out_hbm,
                     didx, rows, sem, acc_sh):
    c = lax.axis_index("c")
    s = lax.axis_index("s")
    wid = c * _NS + s
    base = wid * _EPW
    r0 = wid * _CHUNKS

    pltpu.sync_copy(dst_hbm.at[pl.ds(r0, _CHUNKS)], didx)
    pltpu.sync_copy(zeros_hbm, rows)
    for off, n in _acc_slices():
        pltpu.sync_copy(rows.at[pl.ds(0, n)],
                        acc_sh.at[pl.ds(s * _RPT + off, n)])
    plsc.subcore_barrier()

    def body(i, carry):
        off = base + i * _CH
        pltpu.sync_copy(feat_hbm.at[pl.ds(off, _CH)], rows)
        pltpu.sync_copy(rows, acc_sh.at[didx.at[i]], add=True)
        return carry

    lax.fori_loop(0, _CHUNKS, body, 0)
    plsc.subcore_barrier()

    for off, n in _acc_slices():
        rr = s * _RPT + off
        pltpu.sync_copy(acc_sh.at[pl.ds(rr, n)], rows.at[pl.ds(0, n)])
        pltpu.sync_copy(rows.at[pl.ds(0, n)], out_hbm.at[c, pl.ds(rr, n)])


_seg_sum_ef = pl.kernel(
    _seg_sum_ef_body,
    out_type=jax.ShapeDtypeStruct((_NC, _NPAD, 32), jnp.float32),
    mesh=_mesh,
    compiler_params=pltpu.CompilerParams(use_tc_tiling_on_sc=False),
    scratch_types=[
        pltpu.VMEM((_CHUNKS, _CH), jnp.int32),
        pltpu.VMEM((_CH, 32), jnp.float32),
        pltpu.SemaphoreType.DMA,
        pltpu.VMEM_SHARED((_NPAD, 32), jnp.float32),
    ],
)


def _heads_body(psrc_hbm, pdst_hbm, src_hbm, dst_hbm, out_hbm,
                sidx, didx, acc, sem):
    c = lax.axis_index("c")
    s = lax.axis_index("s")
    wid = c * _NS + s
    base = wid * _EPW
    r0 = wid * _CHUNKS

    pltpu.sync_copy(src_hbm.at[pl.ds(r0, _CHUNKS)], sidx)
    pltpu.sync_copy(dst_hbm.at[pl.ds(r0, _CHUNKS)], didx)

    def body(i, carry):
        pltpu.async_copy(psrc_hbm.at[sidx.at[i]], acc, sem).wait()
        pltpu.async_copy(pdst_hbm.at[didx.at[i]], acc, sem, add=True).wait()
        pltpu.sync_copy(acc, out_hbm.at[pl.ds(base + i * _CH, _CH)])
        return carry

    lax.fori_loop(0, _CHUNKS, body, 0)


_heads = pl.kernel(
    _heads_body,
    out_type=jax.ShapeDtypeStruct((_E, 16), jnp.float32),
    mesh=_mesh,
    compiler_params=pltpu.CompilerParams(use_tc_tiling_on_sc=False),
    scratch_types=[
        pltpu.VMEM((_CHUNKS, _CH), jnp.int32),
        pltpu.VMEM((_CHUNKS, _CH), jnp.int32),
        pltpu.VMEM((_CH, 16), jnp.float32),
        pltpu.SemaphoreType.DMA,
    ],
)


def _dense_body(h_ref, s0_ref, s1_ref, t0_ref, t1_ref,
                wm1_ref, wmaug_ref, wa1_ref, wa2_ref, ba_ref, out_ref):
    f32 = jnp.float32
    S = s0_ref[...] + s1_ref[...]
    T = t0_ref[...] + t1_ref[...]
    hn = jnp.dot(S, wm1_ref[...], preferred_element_type=f32)
    hn = hn + jnp.dot(T, wmaug_ref[...], preferred_element_type=f32)
    acc = jnp.dot(h_ref[...], wa1_ref[...], preferred_element_type=f32)
    acc = acc + jnp.dot(hn, wa2_ref[...], preferred_element_type=f32)
    acc = acc + ba_ref[...]
    out_ref[...] = jnp.where(acc >= 0, acc, 0.01 * acc)


_BR = 1264


def _tc_dense(h, S0, S1, T0, T1, Wm1, WmAug, Wa1, Wa2, ba_row):
    full = lambda shape: pl.BlockSpec(shape, lambda i: (0, 0))
    row = lambda w: pl.BlockSpec((_BR, w), lambda i: (i, 0))
    return pl.pallas_call(
        _dense_body,
        grid=(_NPAD // _BR,),
        in_specs=[row(_D), row(_D), row(_D), row(32), row(32),
                  full((_D, _D)), full((32, _D)), full((_D, _D)),
                  full((_D, _D)), full((1, _D))],
        out_specs=row(_D),
        out_shape=jax.ShapeDtypeStruct((_NPAD, _D), jnp.float32),
    )(h, S0, S1, T0, T1, Wm1, WmAug, Wa1, Wa2, ba_row)


def _proj_body(h_ref, wsrc_ref, wdst_ref, bias_ref, psrc_ref, pdst_ref):
    f32 = jnp.float32
    h = h_ref[...]
    psrc_ref[...] = jnp.dot(h, wsrc_ref[...], preferred_element_type=f32)
    pdst_ref[...] = (jnp.dot(h, wdst_ref[...], preferred_element_type=f32)
                     + bias_ref[...])


def _tc_proj(h, Wsrc, Wdst, bias_row):
    full = lambda shape: pl.BlockSpec(shape, lambda i: (0, 0))
    row = lambda w: pl.BlockSpec((_BR, w), lambda i: (i, 0))
    return pl.pallas_call(
        _proj_body,
        grid=(_NPAD // _BR,),
        in_specs=[row(_D), full((_D, 16)), full((_D, 16)), full((1, 16))],
        out_specs=[row(16), row(16)],
        out_shape=[jax.ShapeDtypeStruct((_NPAD, 16), jnp.float32),
                   jax.ShapeDtypeStruct((_NPAD, 16), jnp.float32)],
    )(h, Wsrc, Wdst, bias_row)


def kernel(nfeats, efeats, edge_index, W1m, b1m, A1, W1a, b1a,
           W2m, b2m, A2, W2a, b2a, Wc, bc, Wf, bf):
    f32 = jnp.float32
    h0 = nfeats[:, 0, :]
    ef = efeats[:, 0, :]
    src = edge_index[0]
    dst = edge_index[1]

    srcp = src.reshape(_NROWS, _CH)
    dstp = dst.reshape(_NROWS, _CH)
    ef32 = jnp.concatenate(
        [ef, jnp.ones((_E, 1), f32), jnp.zeros((_E, 15), f32)], axis=1)
    h0p = jnp.concatenate([h0, jnp.zeros((_NPAD - _N, _D), f32)], axis=0)
    z128 = jnp.zeros((_CH, _D), f32)
    z32 = jnp.zeros((_CH, 32), f32)

    WmAug1 = jnp.concatenate([W1m[_D:], b1m[None], jnp.zeros((15, _D), f32)])
    WmAug2 = jnp.concatenate([W2m[_D:], b2m[None], jnp.zeros((15, _D), f32)])

    Taug = _seg_sum_ef(ef32, srcp, dstp, z32)
    S1 = _seg_sum_h(h0p, srcp, dstp, z128)
    h1 = _tc_dense(h0p, S1[0], S1[1], Taug[0], Taug[1],
                   W1m[:_D], WmAug1, W1a[:_D], W1a[_D:], b1a[None])
    S2 = _seg_sum_h(h1, srcp, dstp, z128)
    h2 = _tc_dense(h1, S2[0], S2[1], Taug[0], Taug[1],
                   W2m[:_D], WmAug2, W2a[:_D], W2a[_D:], b2a[None])

    Wsrc = jnp.concatenate([Wc[:_D], Wf[:_D], jnp.zeros((_D, 4), f32)], axis=1)
    Wdst = jnp.concatenate([Wc[_D:], Wf[_D:], jnp.zeros((_D, 4), f32)], axis=1)
    bias16 = jnp.concatenate([bc, bf, jnp.zeros((4,), f32)])[None]
    Psrc, Pdst = _tc_proj(h2, Wsrc, Wdst, bias16)

    out16 = _heads(Psrc, Pdst, srcp, dstp)
    coarse = out16[:, 0:2]
    fine = out16[:, 2:12]
    return coarse, fine


# speedup vs baseline: 2.2815x; 1.1976x over previous
import functools

import jax
import jax.numpy as jnp
from jax import lax
from jax.experimental import pallas as pl
from jax.experimental.pallas import tpu as pltpu
from jax.experimental.pallas import tpu_sc as plsc

_N = 10000
_E = 320000
_D = 128
_ED = 16

_NC = 2
_NS = 16
_NW = _NC * _NS
_CH = 80
_CHUNKS = 125
_EPW = _CH * _CHUNKS
_NROWS = _E // _CH
_NPAD = 10112
_RPT = _NPAD // _NS

_mesh = plsc.VectorSubcoreMesh(core_axis_name="c", subcore_axis_name="s")


def _acc_slices():
    full, tail = divmod(_RPT, _CH)
    out = [(k * _CH, _CH) for k in range(full)]
    if tail:
        out.append((full * _CH, tail))
    return out


def _seg_sum_h_body(feat_hbm, src_hbm, dst_hbm, zeros_hbm, out_hbm,
                    sa0, sa1, sa2, sa3, da0, da1, da2, da3,
                    rows0, rows1, rows2, rows3,
                    sem_i, sem_g, sem_s, acc_sh):
    c = lax.axis_index("c")
    s = lax.axis_index("s")
    wid = c * _NS + s
    r0 = wid * _CHUNKS

    rows = (rows0, rows1, rows2, rows3)
    sa = (sa0, sa1, sa2, sa3)
    da = (da0, da1, da2, da3)

    pltpu.sync_copy(zeros_hbm, rows0)
    for off, n in _acc_slices():
        pltpu.sync_copy(rows0.at[pl.ds(0, n)],
                        acc_sh.at[pl.ds(s * _RPT + off, n)])
    plsc.subcore_barrier()

    def wait_g():
        pltpu.make_async_copy(zeros_hbm, rows0, sem_g).wait()

    def wait_s():
        pltpu.make_async_copy(zeros_hbm, rows0, sem_s).wait()

    def wait_i():
        pltpu.make_async_copy(src_hbm.at[pl.ds(0, 1)], sa0, sem_i).wait()
        pltpu.make_async_copy(src_hbm.at[pl.ds(0, 1)], da0, sem_i).wait()

    def load_idx(k, j):
        pltpu.async_copy(src_hbm.at[pl.ds(r0 + k, 1)], sa[j], sem_i)
        pltpu.async_copy(dst_hbm.at[pl.ds(r0 + k, 1)], da[j], sem_i)

    def gather(k, j):
        pltpu.async_copy(feat_hbm.at[sa[j].at[0]], rows[j], sem_g)

    def scat(k, j):
        pltpu.async_copy(rows[j], acc_sh.at[da[j].at[0]], sem_s, add=True)

    def step(k, j, n_ahead, n_idx, first=False):
        wait_g()
        scat(k, j)
        if not first:
            wait_s()
        if n_ahead:
            wait_i()
            gather(k + 2, (j + 2) % 4)
        if n_idx:
            load_idx(k + 3, (j + 3) % 4)

    load_idx(0, 0)
    load_idx(1, 1)
    load_idx(2, 2)
    wait_i()
    gather(0, 0)
    wait_i()
    gather(1, 1)
    step(0, 0, True, True, first=True)
    step(1, 1, True, True)
    step(2, 2, True, True)

    def body(i, carry):
        for b4 in range(4):
            k = 3 + 4 * i + b4
            step(k, (3 + b4) % 4, True, True)
        return carry

    lax.fori_loop(0, (_CHUNKS - 9) // 4, body, 0)

    step(_CHUNKS - 6, (_CHUNKS - 6) % 4, True, True)
    step(_CHUNKS - 5, (_CHUNKS - 5) % 4, True, True)
    step(_CHUNKS - 4, (_CHUNKS - 4) % 4, True, True)
    step(_CHUNKS - 3, (_CHUNKS - 3) % 4, True, False)
    step(_CHUNKS - 2, (_CHUNKS - 2) % 4, False, False)
    step(_CHUNKS - 1, (_CHUNKS - 1) % 4, False, False)
    wait_s()
    plsc.subcore_barrier()

    for off, n in _acc_slices():
        rr = s * _RPT + off
        pltpu.sync_copy(acc_sh.at[pl.ds(rr, n)], rows0.at[pl.ds(0, n)])
        pltpu.sync_copy(rows0.at[pl.ds(0, n)], out_hbm.at[c, pl.ds(rr, n)])


_seg_sum_h = pl.kernel(
    _seg_sum_h_body,
    out_type=jax.ShapeDtypeStruct((_NC, _NPAD, _D), jnp.float32),
    mesh=_mesh,
    compiler_params=pltpu.CompilerParams(use_tc_tiling_on_sc=True),
    scratch_types=(
        [pltpu.VMEM((1, _CH), jnp.int32) for _ in range(8)]
        + [pltpu.VMEM((_CH, _D), jnp.float32) for _ in range(4)]
        + [pltpu.SemaphoreType.DMA] * 3
        + [pltpu.VMEM_SHARED((_NPAD, _D), jnp.float32)]
    ),
)


def _seg_sum_ef_body(feat_hbm, src_hbm, dst_hbm, zeros_hbm, out_hbm,
                     didx, rows0, rows1, rows2, rows3, sem_r, sem_s, acc_sh):
    c = lax.axis_index("c")
    s = lax.axis_index("s")
    wid = c * _NS + s
    base = wid * _EPW
    r0 = wid * _CHUNKS

    pltpu.sync_copy(dst_hbm.at[pl.ds(r0, _CHUNKS)], didx)
    pltpu.sync_copy(zeros_hbm, rows0)
    for off, n in _acc_slices():
        pltpu.sync_copy(rows0.at[pl.ds(0, n)],
                        acc_sh.at[pl.ds(s * _RPT + off, n)])
    plsc.subcore_barrier()

    rows = (rows0, rows1, rows2, rows3)

    def wait_r():
        pltpu.make_async_copy(zeros_hbm, rows0, sem_r).wait()

    def wait_s():
        pltpu.make_async_copy(zeros_hbm, rows0, sem_s).wait()

    def read(k, j):
        pltpu.async_copy(feat_hbm.at[pl.ds(base + k * _CH, _CH)],
                         rows[j], sem_r)

    def scat(k, j):
        pltpu.async_copy(rows[j], acc_sh.at[didx.at[k]], sem_s, add=True)

    def step(k, j, n_read, first=False):
        wait_r()
        scat(k, j)
        if not first:
            wait_s()
        if n_read:
            read(k + 3, (j + 3) % 4)

    read(0, 0)
    read(1, 1)
    read(2, 2)
    step(0, 0, True, first=True)

    def body(i, carry):
        for b4 in range(4):
            k = 1 + 4 * i + b4
            step(k, (1 + b4) % 4, True)
        return carry

    lax.fori_loop(0, (_CHUNKS - 5) // 4, body, 0)

    step(_CHUNKS - 4, (_CHUNKS - 4) % 4, True)
    step(_CHUNKS - 3, (_CHUNKS - 3) % 4, False)
    step(_CHUNKS - 2, (_CHUNKS - 2) % 4, False)
    step(_CHUNKS - 1, (_CHUNKS - 1) % 4, False)
    wait_s()
    plsc.subcore_barrier()

    for off, n in _acc_slices():
        rr = s * _RPT + off
        pltpu.sync_copy(acc_sh.at[pl.ds(rr, n)], rows0.at[pl.ds(0, n)])
        pltpu.sync_copy(rows0.at[pl.ds(0, n)], out_hbm.at[c, pl.ds(rr, n)])


_seg_sum_ef = pl.kernel(
    _seg_sum_ef_body,
    out_type=jax.ShapeDtypeStruct((_NC, _NPAD, 32), jnp.float32),
    mesh=_mesh,
    compiler_params=pltpu.CompilerParams(use_tc_tiling_on_sc=False),
    scratch_types=(
        [pltpu.VMEM((_CHUNKS, _CH), jnp.int32)]
        + [pltpu.VMEM((_CH, 32), jnp.float32) for _ in range(4)]
        + [pltpu.SemaphoreType.DMA] * 2
        + [pltpu.VMEM_SHARED((_NPAD, 32), jnp.float32)]
    ),
)


def _heads_body(psrc_hbm, pdst_hbm, src_hbm, dst_hbm, out_hbm,
                sidx, didx, acc0, acc1, acc2, acc3, sem_a, sem_b, sem_c):
    c = lax.axis_index("c")
    s = lax.axis_index("s")
    wid = c * _NS + s
    base = wid * _EPW
    r0 = wid * _CHUNKS

    pltpu.sync_copy(src_hbm.at[pl.ds(r0, _CHUNKS)], sidx)
    pltpu.sync_copy(dst_hbm.at[pl.ds(r0, _CHUNKS)], didx)

    acc = (acc0, acc1, acc2, acc3)

    def wait(sem):
        pltpu.make_async_copy(psrc_hbm.at[pl.ds(0, _CH)], acc0, sem).wait()

    def ga(k, j):
        pltpu.async_copy(psrc_hbm.at[sidx.at[k]], acc[j], sem_a)

    def gb(k, j):
        pltpu.async_copy(pdst_hbm.at[didx.at[k]], acc[j], sem_b, add=True)

    def store(k, j):
        pltpu.async_copy(acc[j], out_hbm.at[pl.ds(base + k * _CH, _CH)],
                         sem_c)

    def step(k, j, n_ahead, first=False):
        wait(sem_a)
        gb(k, j)
        if n_ahead:
            if not first:
                wait(sem_c)
            ga(k + 3, (j + 3) % 4)
        elif not first:
            wait(sem_c)
        wait(sem_b)
        store(k, j)

    ga(0, 0)
    ga(1, 1)
    ga(2, 2)
    step(0, 0, True, first=True)

    def body(i, carry):
        for b4 in range(4):
            k = 1 + 4 * i + b4
            step(k, (1 + b4) % 4, True)
        return carry

    lax.fori_loop(0, (_CHUNKS - 5) // 4, body, 0)

    step(_CHUNKS - 4, (_CHUNKS - 4) % 4, True)
    step(_CHUNKS - 3, (_CHUNKS - 3) % 4, False)
    step(_CHUNKS - 2, (_CHUNKS - 2) % 4, False)
    step(_CHUNKS - 1, (_CHUNKS - 1) % 4, False)
    wait(sem_c)


_heads = pl.kernel(
    _heads_body,
    out_type=jax.ShapeDtypeStruct((_E, 16), jnp.float32),
    mesh=_mesh,
    compiler_params=pltpu.CompilerParams(use_tc_tiling_on_sc=False),
    scratch_types=(
        [pltpu.VMEM((_CHUNKS, _CH), jnp.int32)] * 2
        + [pltpu.VMEM((_CH, 16), jnp.float32) for _ in range(4)]
        + [pltpu.SemaphoreType.DMA] * 3
    ),
)


def _dense_body(h_ref, s0_ref, s1_ref, t0_ref, t1_ref,
                wm1_ref, wmaug_ref, wa1_ref, wa2_ref, ba_ref, out_ref):
    f32 = jnp.float32
    S = s0_ref[...] + s1_ref[...]
    T = t0_ref[...] + t1_ref[...]
    hn = jnp.dot(S, wm1_ref[...], preferred_element_type=f32)
    hn = hn + jnp.dot(T, wmaug_ref[...], preferred_element_type=f32)
    acc = jnp.dot(h_ref[...], wa1_ref[...], preferred_element_type=f32)
    acc = acc + jnp.dot(hn, wa2_ref[...], preferred_element_type=f32)
    acc = acc + ba_ref[...]
    out_ref[...] = jnp.where(acc >= 0, acc, 0.01 * acc)


_BR = 1264


def _tc_dense(h, S0, S1, T0, T1, Wm1, WmAug, Wa1, Wa2, ba_row):
    full = lambda shape: pl.BlockSpec(shape, lambda i: (0, 0))
    row = lambda w: pl.BlockSpec((_BR, w), lambda i: (i, 0))
    return pl.pallas_call(
        _dense_body,
        grid=(_NPAD // _BR,),
        in_specs=[row(_D), row(_D), row(_D), row(32), row(32),
                  full((_D, _D)), full((32, _D)), full((_D, _D)),
                  full((_D, _D)), full((1, _D))],
        out_specs=row(_D),
        out_shape=jax.ShapeDtypeStruct((_NPAD, _D), jnp.float32),
    )(h, S0, S1, T0, T1, Wm1, WmAug, Wa1, Wa2, ba_row)


def _proj_body(h_ref, wsrc_ref, wdst_ref, bias_ref, psrc_ref, pdst_ref):
    f32 = jnp.float32
    h = h_ref[...]
    psrc_ref[...] = jnp.dot(h, wsrc_ref[...], preferred_element_type=f32)
    pdst_ref[...] = (jnp.dot(h, wdst_ref[...], preferred_element_type=f32)
                     + bias_ref[...])


def _tc_proj(h, Wsrc, Wdst, bias_row):
    full = lambda shape: pl.BlockSpec(shape, lambda i: (0, 0))
    row = lambda w: pl.BlockSpec((_BR, w), lambda i: (i, 0))
    return pl.pallas_call(
        _proj_body,
        grid=(_NPAD // _BR,),
        in_specs=[row(_D), full((_D, 16)), full((_D, 16)), full((1, 16))],
        out_specs=[row(16), row(16)],
        out_shape=[jax.ShapeDtypeStruct((_NPAD, 16), jnp.float32),
                   jax.ShapeDtypeStruct((_NPAD, 16), jnp.float32)],
    )(h, Wsrc, Wdst, bias_row)


def kernel(nfeats, efeats, edge_index, W1m, b1m, A1, W1a, b1a,
           W2m, b2m, A2, W2a, b2a, Wc, bc, Wf, bf):
    f32 = jnp.float32
    h0 = nfeats[:, 0, :]
    ef = efeats[:, 0, :]
    src = edge_index[0]
    dst = edge_index[1]

    srcp = src.reshape(_NROWS, _CH)
    dstp = dst.reshape(_NROWS, _CH)
    ef32 = jnp.concatenate(
        [ef, jnp.ones((_E, 1), f32), jnp.zeros((_E, 15), f32)], axis=1)
    h0p = jnp.concatenate([h0, jnp.zeros((_NPAD - _N, _D), f32)], axis=0)
    z128 = jnp.zeros((_CH, _D), f32)
    z32 = jnp.zeros((_CH, 32), f32)

    WmAug1 = jnp.concatenate([W1m[_D:], b1m[None], jnp.zeros((15, _D), f32)])
    WmAug2 = jnp.concatenate([W2m[_D:], b2m[None], jnp.zeros((15, _D), f32)])

    Taug = _seg_sum_ef(ef32, srcp, dstp, z32)
    S1 = _seg_sum_h(h0p, srcp, dstp, z128)
    h1 = _tc_dense(h0p, S1[0], S1[1], Taug[0], Taug[1],
                   W1m[:_D], WmAug1, W1a[:_D], W1a[_D:], b1a[None])
    S2 = _seg_sum_h(h1, srcp, dstp, z128)
    h2 = _tc_dense(h1, S2[0], S2[1], Taug[0], Taug[1],
                   W2m[:_D], WmAug2, W2a[:_D], W2a[_D:], b2a[None])

    Wsrc = jnp.concatenate([Wc[:_D], Wf[:_D], jnp.zeros((_D, 4), f32)], axis=1)
    Wdst = jnp.concatenate([Wc[_D:], Wf[_D:], jnp.zeros((_D, 4), f32)], axis=1)
    bias16 = jnp.concatenate([bc, bf, jnp.zeros((4,), f32)])[None]
    Psrc, Pdst = _tc_proj(h2, Wsrc, Wdst, bias16)

    out16 = _heads(Psrc, Pdst, srcp, dstp)
    coarse = out16[:, 0:2]
    fine = out16[:, 2:12]
    return coarse, fine
